# baseline (speedup 1.0000x reference)
import functools
import jax
import jax.numpy as jnp
from jax import lax
from jax.experimental import pallas as pl
from jax.experimental.pallas import tpu as pltpu
from jax.experimental.pallas import tpu_sc as plsc

_N = 100000
_C = 1000
_NB = 15

_SC_CHUNK = 40
_R_SC = 32000
_ROWS_PER_TILE = _R_SC // 32
_N_CHUNKS = _ROWS_PER_TILE // _SC_CHUNK
_TC_ROWS = _N - _R_SC
_TC_BLK = 2000
_NVEC = _C // 16
_TAIL = _C - _NVEC * 16


def _shuf(v, idx):
    dnums = lax.GatherDimensionNumbers(
        offset_dims=(), collapsed_slice_dims=(0,), start_index_map=(0,)
    )
    return lax.gather(
        v, idx[:, None], dnums, slice_sizes=(1,),
        mode=lax.GatherScatterMode.PROMISE_IN_BOUNDS,
    )


def _tc_kernel(x_ref, lab_ref, out_ref, acc_ref):
    i = pl.program_id(0)
    nsteps = pl.num_programs(0)

    @pl.when(i == 0)
    def _():
        acc_ref[...] = jnp.zeros_like(acc_ref)

    x = x_ref[...]
    lab = lab_ref[...]
    m = jnp.max(x, axis=1, keepdims=True)
    s = jnp.sum(jnp.exp(x - m), axis=1, keepdims=True)
    conf = 1.0 / s
    pred = jnp.argmax(x, axis=1)[:, None].astype(jnp.int32)
    accf = (pred == lab).astype(jnp.float32)
    d = conf - accf
    boundaries = (
        lax.broadcasted_iota(jnp.int32, (1, _NB + 1), 1).astype(jnp.float32) / _NB
    )
    mask = (conf > boundaries).astype(jnp.float32)
    acc_ref[...] += jnp.sum(d * mask, axis=0, keepdims=True)

    @pl.when(i == nsteps - 1)
    def _():
        out_ref[...] = acc_ref[...]


def _sc_body(x_hbm, lab_hbm, out_hbm, buf, labbuf, tvec):
    wid = lax.axis_index("s") * 2 + lax.axis_index("c")
    base_row = wid * _ROWS_PER_TILE
    lanes = lax.iota(jnp.int32, 16)
    valid_tail = lanes < _TAIL
    bounds = lanes.astype(jnp.float32) / _NB
    neg_inf = jnp.full((16,), -jnp.inf, dtype=jnp.float32)
    zeros16 = jnp.zeros((16,), dtype=jnp.float32)
    big = jnp.full((16,), jnp.int32(2**30), dtype=jnp.int32)

    def chunk_body(ci, t):
        row0 = base_row + ci * _SC_CHUNK
        pltpu.sync_copy(
            x_hbm.at[pl.ds(row0 * _C, _SC_CHUNK * _C)],
            buf.at[pl.ds(0, _SC_CHUNK * _C)],
        )
        pltpu.sync_copy(lab_hbm.at[pl.ds(row0, _SC_CHUNK)], labbuf.at[pl.ds(0, _SC_CHUNK)])

        def row_body(r, t):
            off = r * _C

            def max_body(j, mv):
                v = buf[pl.ds(off + j * 16, 16)]
                return jnp.maximum(mv, v)

            mv = lax.fori_loop(0, _NVEC, max_body, neg_inf)
            vt = buf[pl.ds(off + _NVEC * 16, 16)]
            mv = jnp.maximum(mv, jnp.where(valid_tail, vt, neg_inf))
            for sh in (8, 4, 2, 1):
                mv = jnp.maximum(mv, _shuf(mv, lanes ^ sh))
            msplat = mv

            def sum_body(j, carry):
                sv, iv = carry
                v = buf[pl.ds(off + j * 16, 16)]
                sv = sv + jnp.exp(v - msplat)
                cand = jnp.where(v == msplat, lanes + j * 16, big)
                return sv, jnp.minimum(iv, cand)

            sv, iv = lax.fori_loop(0, _NVEC, sum_body, (zeros16, big))
            et = jnp.where(valid_tail, jnp.exp(vt - msplat), zeros16)
            sv = sv + et
            candt = jnp.where(
                (vt == msplat) & valid_tail, lanes + _NVEC * 16, big
            )
            iv = jnp.minimum(iv, candt)
            for sh in (8, 4, 2, 1):
                perm = lanes ^ sh
                sv = sv + _shuf(sv, perm)
                iv = jnp.minimum(iv, _shuf(iv, perm))
            conf = 1.0 / sv
            lab = labbuf[pl.ds(r, 16)][0]
            labsplat = jnp.full((16,), lab, dtype=jnp.int32)
            accf = jnp.where(iv == labsplat, 1.0, 0.0).astype(jnp.float32)
            d = conf - accf
            return t + jnp.where(conf > bounds, d, zeros16)

        return lax.fori_loop(0, _SC_CHUNK, row_body, t)

    t = lax.fori_loop(0, _N_CHUNKS, chunk_body, jnp.zeros((16,), jnp.float32))
    tvec[...] = t
    pltpu.sync_copy(tvec, out_hbm.at[pl.ds(wid * 16, 16)])


def _combine_kernel(ttc_ref, tsc_ref, out_ref):
    t = ttc_ref[...] + jnp.sum(tsc_ref[...], axis=0, keepdims=True)
    gaps = jnp.abs(t[:, :_NB] - t[:, 1 : _NB + 1])
    out_ref[...] = jnp.sum(gaps, axis=1, keepdims=True) / _N


@functools.partial(jax.jit)
def kernel(logits, labels):
    labels = labels.astype(jnp.int32)

    t_tc = pl.pallas_call(
        _tc_kernel,
        grid=(_TC_ROWS // _TC_BLK,),
        in_specs=[
            pl.BlockSpec((_TC_BLK, _C), lambda i: (i + _R_SC // _TC_BLK, 0)),
            pl.BlockSpec((_TC_BLK, 1), lambda i: (i + _R_SC // _TC_BLK, 0)),
        ],
        out_specs=pl.BlockSpec((1, _NB + 1), lambda i: (0, 0)),
        out_shape=jax.ShapeDtypeStruct((1, _NB + 1), jnp.float32),
        scratch_shapes=[pltpu.VMEM((1, _NB + 1), jnp.float32)],
    )(logits, labels.reshape(_N, 1))

    sc_fn = functools.partial(
        pl.kernel,
        mesh=plsc.VectorSubcoreMesh(core_axis_name="c", subcore_axis_name="s"),
        out_type=jax.ShapeDtypeStruct((32 * 16,), jnp.float32),
        scratch_types=[
            pltpu.VMEM((_SC_CHUNK * _C + 16,), jnp.float32),
            pltpu.VMEM((_SC_CHUNK + 16,), jnp.int32),
            pltpu.VMEM((16,), jnp.float32),
        ],
    )(_sc_body)
    t_sc = sc_fn(logits.reshape(_N * _C), labels)

    out = pl.pallas_call(
        _combine_kernel,
        in_specs=[
            pl.BlockSpec((1, _NB + 1), lambda: (0, 0)),
            pl.BlockSpec((32, 16), lambda: (0, 0)),
        ],
        out_specs=pl.BlockSpec((1, 1), lambda: (0, 0)),
        out_shape=jax.ShapeDtypeStruct((1, 1), jnp.float32),
    )(t_tc, t_sc.reshape(32, 16))
    return out.reshape(1)


# speedup vs baseline: 2.1610x; 2.1610x over previous
import functools
import jax
import jax.numpy as jnp
from jax import lax
from jax.experimental import pallas as pl
from jax.experimental.pallas import tpu as pltpu
from jax.experimental.pallas import tpu_sc as plsc

_N = 100000
_C = 1000
_NB = 15

_SC_CHUNK = 40
_R_SC = 12800
_ROWS_PER_TILE = _R_SC // 32
_N_CHUNKS = _ROWS_PER_TILE // _SC_CHUNK
_TC_ROWS = _N - _R_SC
_TC_BLK = 4000
_NVEC = _C // 16
_TAIL = _C - _NVEC * 16


def _shuf(v, idx):
    dnums = lax.GatherDimensionNumbers(
        offset_dims=(), collapsed_slice_dims=(0,), start_index_map=(0,)
    )
    return lax.gather(
        v, idx[:, None], dnums, slice_sizes=(1,),
        mode=lax.GatherScatterMode.PROMISE_IN_BOUNDS,
    )


def _tc_kernel(x_ref, lab_ref, out_ref, acc_ref):
    i = pl.program_id(0)
    nsteps = pl.num_programs(0)

    @pl.when(i == 0)
    def _():
        acc_ref[...] = jnp.zeros_like(acc_ref)

    x = x_ref[...]
    lab = lab_ref[0, 0, :]
    rowid = i * _TC_BLK + lax.broadcasted_iota(jnp.int32, (_TC_BLK, 1), 0)
    rmask = (rowid >= _R_SC).astype(jnp.float32)
    m = jnp.max(x, axis=1, keepdims=True)
    s = jnp.sum(jnp.exp(x - m), axis=1, keepdims=True)
    conf = 1.0 / s
    pred = jnp.argmax(x, axis=1).astype(jnp.int32)
    accf = (pred == lab).astype(jnp.float32)[:, None]
    d = (conf - accf) * rmask
    boundaries = (
        lax.broadcasted_iota(jnp.int32, (1, _NB + 1), 1).astype(jnp.float32) / _NB
    )
    mask = (conf > boundaries).astype(jnp.float32)
    acc_ref[...] += jnp.sum(d * mask, axis=0, keepdims=True)

    @pl.when(i == nsteps - 1)
    def _():
        out_ref[...] = acc_ref[...]


def _sc_body(x_hbm, lab_hbm, out_hbm, buf, labbuf, tvec):
    wid = lax.axis_index("s") * 2 + lax.axis_index("c")
    base_row = wid * _ROWS_PER_TILE
    lanes = lax.iota(jnp.int32, 16)
    bounds = lanes.astype(jnp.float32) / _NB
    neg_inf = jnp.full((16,), -jnp.inf, dtype=jnp.float32)
    zeros16 = jnp.zeros((16,), dtype=jnp.float32)
    big = jnp.full((16,), jnp.int32(2**30), dtype=jnp.int32)
    R = 4

    def chunk_body(ci, t):
        row0 = base_row + ci * _SC_CHUNK
        pltpu.sync_copy(x_hbm.at[pl.ds(row0, _SC_CHUNK)], buf)
        pltpu.sync_copy(lab_hbm.at[pl.ds(row0, _SC_CHUNK)], labbuf.at[pl.ds(0, _SC_CHUNK)])

        def group_body(g, t):
            rbase = g * R

            def max_body(j, mvs):
                return tuple(
                    jnp.maximum(mvs[k], buf.at[rbase + k][pl.ds(j * 16, 16)])
                    for k in range(R)
                )

            mvs = lax.fori_loop(0, _NVEC, max_body, (neg_inf,) * R)
            vts = [buf.at[rbase + k][pl.ds(_C - 16, 16)] for k in range(R)]
            mvs = [
                jnp.maximum(mvs[k], jnp.where(lanes >= 16 - _TAIL, vts[k], neg_inf))
                for k in range(R)
            ]
            for sh in (8, 4, 2, 1):
                mvs = [jnp.maximum(mv, _shuf(mv, lanes ^ sh)) for mv in mvs]

            def sum_body(j, carry):
                svs, ivs = carry
                idx = lanes + j * 16
                nsvs, nivs = [], []
                for k in range(R):
                    v = buf.at[rbase + k][pl.ds(j * 16, 16)]
                    nsvs.append(svs[k] + jnp.exp(v - mvs[k]))
                    nivs.append(
                        jnp.minimum(ivs[k], jnp.where(v == mvs[k], idx, big))
                    )
                return tuple(nsvs), tuple(nivs)

            svs, ivs = lax.fori_loop(
                0, _NVEC, sum_body, ((zeros16,) * R, (big,) * R)
            )
            idxt = lanes + _C - 16
            svs = [
                svs[k]
                + jnp.where(lanes >= 16 - _TAIL, jnp.exp(vts[k] - mvs[k]), zeros16)
                for k in range(R)
            ]
            ivs = [
                jnp.minimum(
                    ivs[k],
                    jnp.where((vts[k] == mvs[k]) & (lanes >= 16 - _TAIL), idxt, big),
                )
                for k in range(R)
            ]
            for sh in (8, 4, 2, 1):
                svs = [sv + _shuf(sv, lanes ^ sh) for sv in svs]
                ivs = [jnp.minimum(iv, _shuf(iv, lanes ^ sh)) for iv in ivs]
            labv = labbuf[pl.ds(g * R, 16)]
            for k in range(R):
                conf = 1.0 / svs[k]
                labsplat = jnp.full((16,), labv[k], dtype=jnp.int32)
                accf = jnp.where(ivs[k] == labsplat, 1.0, 0.0).astype(
                    jnp.float32
                )
                d = conf - accf
                t = t + jnp.where(conf > bounds, d, zeros16)
            return t

        return lax.fori_loop(0, _SC_CHUNK // R, group_body, t)

    t = lax.fori_loop(0, _N_CHUNKS, chunk_body, jnp.zeros((16,), jnp.float32))
    tvec[...] = t
    pltpu.sync_copy(tvec, out_hbm.at[pl.ds(wid * 16, 16)])


def _combine_kernel(ttc_ref, tsc_ref, out_ref):
    t = ttc_ref[...] + jnp.sum(tsc_ref[...], axis=0, keepdims=True)
    gaps = jnp.abs(t[:, :_NB] - t[:, 1 : _NB + 1])
    out_ref[...] = jnp.sum(gaps, axis=1, keepdims=True) / _N


@functools.partial(jax.jit)
def kernel(logits, labels):
    labels = labels.astype(jnp.int32)

    sc_fn = functools.partial(
        pl.kernel,
        mesh=plsc.VectorSubcoreMesh(core_axis_name="c", subcore_axis_name="s"),
        compiler_params=pltpu.CompilerParams(use_tc_tiling_on_sc=False),
        out_type=jax.ShapeDtypeStruct((32 * 16,), jnp.float32),
        scratch_types=[
            pltpu.VMEM((_SC_CHUNK, _C), jnp.float32),
            pltpu.VMEM((_SC_CHUNK + 16,), jnp.int32),
            pltpu.VMEM((16,), jnp.float32),
        ],
    )(_sc_body)
    t_sc = sc_fn(logits[:_R_SC], labels[:_R_SC])

    t_tc = pl.pallas_call(
        _tc_kernel,
        grid=(_N // _TC_BLK,),
        in_specs=[
            pl.BlockSpec((_TC_BLK, _C), lambda i: (i, 0)),
            pl.BlockSpec((1, 1, _TC_BLK), lambda i: (i, 0, 0)),
        ],
        out_specs=pl.BlockSpec((1, _NB + 1), lambda i: (0, 0)),
        out_shape=jax.ShapeDtypeStruct((1, _NB + 1), jnp.float32),
        scratch_shapes=[pltpu.VMEM((1, _NB + 1), jnp.float32)],
    )(logits, labels.reshape(_N // _TC_BLK, 1, _TC_BLK))

    out = pl.pallas_call(
        _combine_kernel,
        in_specs=[
            pl.BlockSpec((1, _NB + 1), lambda: (0, 0)),
            pl.BlockSpec((32, 16), lambda: (0, 0)),
        ],
        out_specs=pl.BlockSpec((1, 1), lambda: (0, 0)),
        out_shape=jax.ShapeDtypeStruct((1, 1), jnp.float32),
    )(t_tc, t_sc.reshape(32, 16))
    return out.reshape(1)


# speedup vs baseline: 2.4717x; 1.1438x over previous
import functools
import jax
import jax.numpy as jnp
from jax import lax
from jax.experimental import pallas as pl
from jax.experimental.pallas import tpu as pltpu
from jax.experimental.pallas import tpu_sc as plsc

_N = 100000
_C = 1000
_NB = 15

_SC_CHUNK = 40
_R_SC = 2560
_ROWS_PER_TILE = _R_SC // 32
_N_CHUNKS = _ROWS_PER_TILE // _SC_CHUNK
_TC_ROWS = _N - _R_SC
_TC_BLK = 4000
_NVEC = _C // 16
_TAIL = _C - _NVEC * 16


def _shuf(v, idx):
    dnums = lax.GatherDimensionNumbers(
        offset_dims=(), collapsed_slice_dims=(0,), start_index_map=(0,)
    )
    return lax.gather(
        v, idx[:, None], dnums, slice_sizes=(1,),
        mode=lax.GatherScatterMode.PROMISE_IN_BOUNDS,
    )


def _tc_kernel(x_ref, lab_ref, out_ref, acc_ref):
    i = pl.program_id(0)
    nsteps = pl.num_programs(0)

    @pl.when(i == 0)
    def _():
        acc_ref[...] = jnp.zeros_like(acc_ref)

    x = x_ref[...]
    lab = lab_ref[0, 0, :]
    rowid = i * _TC_BLK + lax.broadcasted_iota(jnp.int32, (_TC_BLK, 1), 0)
    rmask = (rowid >= _R_SC).astype(jnp.float32)
    m = jnp.max(x, axis=1, keepdims=True)
    s = jnp.sum(jnp.exp(x - m), axis=1, keepdims=True)
    conf = 1.0 / s
    pred = jnp.argmax(x, axis=1).astype(jnp.int32)
    accf = (pred == lab).astype(jnp.float32)[:, None]
    d = (conf - accf) * rmask
    boundaries = (
        lax.broadcasted_iota(jnp.int32, (1, _NB + 1), 1).astype(jnp.float32) / _NB
    )
    mask = (conf > boundaries).astype(jnp.float32)
    acc_ref[...] += jnp.sum(d * mask, axis=0, keepdims=True)

    @pl.when(i == nsteps - 1)
    def _():
        out_ref[...] = acc_ref[...]


def _sc_body(x_hbm, lab_hbm, out_hbm, buf, labbuf, tvec):
    wid = lax.axis_index("s") * 2 + lax.axis_index("c")
    base_row = wid * _ROWS_PER_TILE
    lanes = lax.iota(jnp.int32, 16)
    bounds = lanes.astype(jnp.float32) / _NB
    neg_inf = jnp.full((16,), -jnp.inf, dtype=jnp.float32)
    zeros16 = jnp.zeros((16,), dtype=jnp.float32)
    big = jnp.full((16,), jnp.int32(2**30), dtype=jnp.int32)
    R = 4

    def chunk_body(ci, t):
        row0 = base_row + ci * _SC_CHUNK
        pltpu.sync_copy(x_hbm.at[pl.ds(row0, _SC_CHUNK)], buf)
        pltpu.sync_copy(lab_hbm.at[pl.ds(row0, _SC_CHUNK)], labbuf.at[pl.ds(0, _SC_CHUNK)])

        def group_body(g, t):
            rbase = g * R

            def max_body(j, mvs):
                return tuple(
                    jnp.maximum(mvs[k], buf.at[rbase + k][pl.ds(j * 16, 16)])
                    for k in range(R)
                )

            mvs = lax.fori_loop(0, _NVEC, max_body, (neg_inf,) * R)
            vts = [buf.at[rbase + k][pl.ds(_C - 16, 16)] for k in range(R)]
            mvs = [
                jnp.maximum(mvs[k], jnp.where(lanes >= 16 - _TAIL, vts[k], neg_inf))
                for k in range(R)
            ]
            for sh in (8, 4, 2, 1):
                mvs = [jnp.maximum(mv, _shuf(mv, lanes ^ sh)) for mv in mvs]

            def sum_body(j, carry):
                svs, ivs = carry
                idx = lanes + j * 16
                nsvs, nivs = [], []
                for k in range(R):
                    v = buf.at[rbase + k][pl.ds(j * 16, 16)]
                    nsvs.append(svs[k] + jnp.exp(v - mvs[k]))
                    nivs.append(
                        jnp.minimum(ivs[k], jnp.where(v == mvs[k], idx, big))
                    )
                return tuple(nsvs), tuple(nivs)

            svs, ivs = lax.fori_loop(
                0, _NVEC, sum_body, ((zeros16,) * R, (big,) * R)
            )
            idxt = lanes + _C - 16
            svs = [
                svs[k]
                + jnp.where(lanes >= 16 - _TAIL, jnp.exp(vts[k] - mvs[k]), zeros16)
                for k in range(R)
            ]
            ivs = [
                jnp.minimum(
                    ivs[k],
                    jnp.where((vts[k] == mvs[k]) & (lanes >= 16 - _TAIL), idxt, big),
                )
                for k in range(R)
            ]
            for sh in (8, 4, 2, 1):
                svs = [sv + _shuf(sv, lanes ^ sh) for sv in svs]
                ivs = [jnp.minimum(iv, _shuf(iv, lanes ^ sh)) for iv in ivs]
            labv = labbuf[pl.ds(g * R, 16)]
            for k in range(R):
                conf = 1.0 / svs[k]
                labsplat = jnp.full((16,), labv[k], dtype=jnp.int32)
                accf = jnp.where(ivs[k] == labsplat, 1.0, 0.0).astype(
                    jnp.float32
                )
                d = conf - accf
                t = t + jnp.where(conf > bounds, d, zeros16)
            return t

        return lax.fori_loop(0, _SC_CHUNK // R, group_body, t)

    t = lax.fori_loop(0, _N_CHUNKS, chunk_body, jnp.zeros((16,), jnp.float32))
    tvec[...] = t
    pltpu.sync_copy(tvec, out_hbm.at[pl.ds(wid * 16, 16)])


def _combine_kernel(ttc_ref, tsc_ref, out_ref):
    t = ttc_ref[...] + jnp.sum(tsc_ref[...], axis=0, keepdims=True)
    gaps = jnp.abs(t[:, :_NB] - t[:, 1 : _NB + 1])
    out_ref[...] = jnp.sum(gaps, axis=1, keepdims=True) / _N


@functools.partial(jax.jit)
def kernel(logits, labels):
    labels = labels.astype(jnp.int32)

    sc_fn = functools.partial(
        pl.kernel,
        mesh=plsc.VectorSubcoreMesh(core_axis_name="c", subcore_axis_name="s"),
        compiler_params=pltpu.CompilerParams(use_tc_tiling_on_sc=False),
        out_type=jax.ShapeDtypeStruct((32 * 16,), jnp.float32),
        scratch_types=[
            pltpu.VMEM((_SC_CHUNK, _C), jnp.float32),
            pltpu.VMEM((_SC_CHUNK + 16,), jnp.int32),
            pltpu.VMEM((16,), jnp.float32),
        ],
    )(_sc_body)
    t_sc = sc_fn(logits[:_R_SC], labels[:_R_SC])

    t_tc = pl.pallas_call(
        _tc_kernel,
        grid=(_N // _TC_BLK,),
        in_specs=[
            pl.BlockSpec((_TC_BLK, _C), lambda i: (i, 0)),
            pl.BlockSpec((1, 1, _TC_BLK), lambda i: (i, 0, 0)),
        ],
        out_specs=pl.BlockSpec((1, _NB + 1), lambda i: (0, 0)),
        out_shape=jax.ShapeDtypeStruct((1, _NB + 1), jnp.float32),
        scratch_shapes=[pltpu.VMEM((1, _NB + 1), jnp.float32)],
    )(logits, labels.reshape(_N // _TC_BLK, 1, _TC_BLK))

    out = pl.pallas_call(
        _combine_kernel,
        in_specs=[
            pl.BlockSpec((1, _NB + 1), lambda: (0, 0)),
            pl.BlockSpec((32, 16), lambda: (0, 0)),
        ],
        out_specs=pl.BlockSpec((1, 1), lambda: (0, 0)),
        out_shape=jax.ShapeDtypeStruct((1, 1), jnp.float32),
    )(t_tc, t_sc.reshape(32, 16))
    return out.reshape(1)


# speedup vs baseline: 2.4736x; 1.0008x over previous
import functools
import jax
import jax.numpy as jnp
from jax import lax
from jax.experimental import pallas as pl
from jax.experimental.pallas import tpu as pltpu
from jax.experimental.pallas import tpu_sc as plsc

_N = 100000
_C = 1000
_NB = 15

_SC_CHUNK = 40
_R_SC = 1280
_ROWS_PER_TILE = _R_SC // 32
_N_CHUNKS = _ROWS_PER_TILE // _SC_CHUNK
_TC_ROWS = _N - _R_SC
_TC_BLK = 4000
_NVEC = _C // 16
_TAIL = _C - _NVEC * 16


def _shuf(v, idx):
    dnums = lax.GatherDimensionNumbers(
        offset_dims=(), collapsed_slice_dims=(0,), start_index_map=(0,)
    )
    return lax.gather(
        v, idx[:, None], dnums, slice_sizes=(1,),
        mode=lax.GatherScatterMode.PROMISE_IN_BOUNDS,
    )


def _tc_kernel(x_ref, lab_ref, out_ref, acc_ref):
    i = pl.program_id(0)
    nsteps = pl.num_programs(0)

    @pl.when(i == 0)
    def _():
        acc_ref[...] = jnp.zeros_like(acc_ref)

    x = x_ref[...]
    lab = lab_ref[0, 0, :]
    rowid = i * _TC_BLK + lax.broadcasted_iota(jnp.int32, (_TC_BLK, 1), 0)
    rmask = (rowid >= _R_SC).astype(jnp.float32)
    m = jnp.max(x, axis=1, keepdims=True)
    s = jnp.sum(jnp.exp(x - m), axis=1, keepdims=True)
    conf = 1.0 / s
    pred = jnp.argmax(x, axis=1).astype(jnp.int32)
    accf = (pred == lab).astype(jnp.float32)[:, None]
    d = (conf - accf) * rmask
    boundaries = (
        lax.broadcasted_iota(jnp.int32, (1, _NB + 1), 1).astype(jnp.float32) / _NB
    )
    mask = (conf > boundaries).astype(jnp.float32)
    acc_ref[...] += jnp.sum(d * mask, axis=0, keepdims=True)

    @pl.when(i == nsteps - 1)
    def _():
        out_ref[...] = acc_ref[...]


def _sc_body(x_hbm, lab_hbm, out_hbm, buf, labbuf, tvec):
    wid = lax.axis_index("s") * 2 + lax.axis_index("c")
    base_row = wid * _ROWS_PER_TILE
    lanes = lax.iota(jnp.int32, 16)
    bounds = lanes.astype(jnp.float32) / _NB
    neg_inf = jnp.full((16,), -jnp.inf, dtype=jnp.float32)
    zeros16 = jnp.zeros((16,), dtype=jnp.float32)
    big = jnp.full((16,), jnp.int32(2**30), dtype=jnp.int32)
    R = 4

    def chunk_body(ci, t):
        row0 = base_row + ci * _SC_CHUNK
        pltpu.sync_copy(x_hbm.at[pl.ds(row0, _SC_CHUNK)], buf)
        pltpu.sync_copy(lab_hbm.at[pl.ds(row0, _SC_CHUNK)], labbuf.at[pl.ds(0, _SC_CHUNK)])

        def group_body(g, t):
            rbase = g * R

            def max_body(j, mvs):
                return tuple(
                    jnp.maximum(mvs[k], buf.at[rbase + k][pl.ds(j * 16, 16)])
                    for k in range(R)
                )

            mvs = lax.fori_loop(0, _NVEC, max_body, (neg_inf,) * R)
            vts = [buf.at[rbase + k][pl.ds(_C - 16, 16)] for k in range(R)]
            mvs = [
                jnp.maximum(mvs[k], jnp.where(lanes >= 16 - _TAIL, vts[k], neg_inf))
                for k in range(R)
            ]
            for sh in (8, 4, 2, 1):
                mvs = [jnp.maximum(mv, _shuf(mv, lanes ^ sh)) for mv in mvs]

            def sum_body(j, carry):
                svs, ivs = carry
                idx = lanes + j * 16
                nsvs, nivs = [], []
                for k in range(R):
                    v = buf.at[rbase + k][pl.ds(j * 16, 16)]
                    nsvs.append(svs[k] + jnp.exp(v - mvs[k]))
                    nivs.append(
                        jnp.minimum(ivs[k], jnp.where(v == mvs[k], idx, big))
                    )
                return tuple(nsvs), tuple(nivs)

            svs, ivs = lax.fori_loop(
                0, _NVEC, sum_body, ((zeros16,) * R, (big,) * R)
            )
            idxt = lanes + _C - 16
            svs = [
                svs[k]
                + jnp.where(lanes >= 16 - _TAIL, jnp.exp(vts[k] - mvs[k]), zeros16)
                for k in range(R)
            ]
            ivs = [
                jnp.minimum(
                    ivs[k],
                    jnp.where((vts[k] == mvs[k]) & (lanes >= 16 - _TAIL), idxt, big),
                )
                for k in range(R)
            ]
            for sh in (8, 4, 2, 1):
                svs = [sv + _shuf(sv, lanes ^ sh) for sv in svs]
                ivs = [jnp.minimum(iv, _shuf(iv, lanes ^ sh)) for iv in ivs]
            labv = labbuf[pl.ds(g * R, 16)]
            for k in range(R):
                conf = 1.0 / svs[k]
                labsplat = jnp.full((16,), labv[k], dtype=jnp.int32)
                accf = jnp.where(ivs[k] == labsplat, 1.0, 0.0).astype(
                    jnp.float32
                )
                d = conf - accf
                t = t + jnp.where(conf > bounds, d, zeros16)
            return t

        return lax.fori_loop(0, _SC_CHUNK // R, group_body, t)

    t = lax.fori_loop(0, _N_CHUNKS, chunk_body, jnp.zeros((16,), jnp.float32))
    tvec[...] = t
    pltpu.sync_copy(tvec, out_hbm.at[pl.ds(wid * 16, 16)])


def _combine_kernel(ttc_ref, tsc_ref, out_ref):
    t = ttc_ref[...] + jnp.sum(tsc_ref[...], axis=0, keepdims=True)
    gaps = jnp.abs(t[:, :_NB] - t[:, 1 : _NB + 1])
    out_ref[...] = jnp.sum(gaps, axis=1, keepdims=True) / _N


@functools.partial(jax.jit)
def kernel(logits, labels):
    labels = labels.astype(jnp.int32)

    sc_fn = functools.partial(
        pl.kernel,
        mesh=plsc.VectorSubcoreMesh(core_axis_name="c", subcore_axis_name="s"),
        compiler_params=pltpu.CompilerParams(use_tc_tiling_on_sc=False),
        out_type=jax.ShapeDtypeStruct((32 * 16,), jnp.float32),
        scratch_types=[
            pltpu.VMEM((_SC_CHUNK, _C), jnp.float32),
            pltpu.VMEM((_SC_CHUNK + 16,), jnp.int32),
            pltpu.VMEM((16,), jnp.float32),
        ],
    )(_sc_body)
    t_sc = sc_fn(logits[:_R_SC], labels[:_R_SC])

    t_tc = pl.pallas_call(
        _tc_kernel,
        grid=(_N // _TC_BLK,),
        in_specs=[
            pl.BlockSpec((_TC_BLK, _C), lambda i: (i, 0)),
            pl.BlockSpec((1, 1, _TC_BLK), lambda i: (i, 0, 0)),
        ],
        out_specs=pl.BlockSpec((1, _NB + 1), lambda i: (0, 0)),
        out_shape=jax.ShapeDtypeStruct((1, _NB + 1), jnp.float32),
        scratch_shapes=[pltpu.VMEM((1, _NB + 1), jnp.float32)],
    )(logits, labels.reshape(_N // _TC_BLK, 1, _TC_BLK))

    out = pl.pallas_call(
        _combine_kernel,
        in_specs=[
            pl.BlockSpec((1, _NB + 1), lambda: (0, 0)),
            pl.BlockSpec((32, 16), lambda: (0, 0)),
        ],
        out_specs=pl.BlockSpec((1, 1), lambda: (0, 0)),
        out_shape=jax.ShapeDtypeStruct((1, 1), jnp.float32),
    )(t_tc, t_sc.reshape(32, 16))
    return out.reshape(1)
